# baseline (device time: 68515 ns/iter reference)
import jax
import jax.numpy as jnp
from jax import lax
from jax.experimental import pallas as pl
from jax.experimental.pallas import tpu as pltpu

Z = 4
AW = 128


CAP = 128


def kernel(x, assign, W1, W2):
    T, D = x.shape
    E, _, F = W1.shape
    CW = D + AW

    order = jnp.argsort(assign)
    xb = jnp.take(x, order, axis=0).astype(jnp.bfloat16)
    ab = jnp.take(assign, order).reshape(T, 1).astype(jnp.bfloat16)
    w1b = W1.astype(jnp.bfloat16)
    w2b = W2.astype(jnp.bfloat16)

    def body(x_ref, a_ref, w1_ref, w2_ref, out_ref,
             comm, pbuf, rbuf, ag_send, ag_recv, rs_send, rs_recv):
        mx = lax.axis_index("x")
        my = lax.axis_index("y")
        mz = lax.axis_index("z")
        right = lax.rem(mz + 1, Z)
        left = lax.rem(mz + Z - 1, Z)

        barrier = pltpu.get_barrier_semaphore()
        for d in range(1, Z):
            peer = lax.rem(mz + d, Z)
            pl.semaphore_signal(
                barrier, inc=1,
                device_id=(mx, my, peer),
                device_id_type=pl.DeviceIdType.MESH,
            )
        pl.semaphore_wait(barrier, Z - 1)

        comm[0, :, :D] = x_ref[...]
        comm[0, :, D:] = jnp.broadcast_to(a_ref[...], (T, AW))

        ag_a = pltpu.make_async_remote_copy(
            src_ref=comm.at[0], dst_ref=comm.at[Z - 1],
            send_sem=ag_send.at[0], recv_sem=ag_recv.at[0],
            device_id=(mx, my, right), device_id_type=pl.DeviceIdType.MESH,
        )
        ag_b = pltpu.make_async_remote_copy(
            src_ref=comm.at[0], dst_ref=comm.at[1],
            send_sem=ag_send.at[1], recv_sem=ag_recv.at[1],
            device_id=(mx, my, left), device_id_type=pl.DeviceIdType.MESH,
        )
        ag_c = pltpu.make_async_remote_copy(
            src_ref=comm.at[Z - 1], dst_ref=comm.at[Z - 2],
            send_sem=ag_send.at[2], recv_sem=ag_recv.at[2],
            device_id=(mx, my, right), device_id_type=pl.DeviceIdType.MESH,
        )

        w1s = [w1_ref[e] for e in range(E)]
        w2s = [w2_ref[e] for e in range(E)]

        def expert_block(d):
            av_full = comm[d, :, D:D + 1]
            pbuf[d] = jnp.zeros((T, D), jnp.bfloat16)
            for k in range(E):
                eb = (E * mz + k).astype(jnp.bfloat16)
                below = jnp.sum((av_full < eb).astype(jnp.int32))
                start = jnp.minimum((below // 16) * 16, T - CAP)
                xw = comm[d, pl.ds(start, CAP), :D]
                aw = comm[d, pl.ds(start, CAP), D:D + 1]
                xm = jnp.where(aw == eb, xw, jnp.bfloat16(0))
                h1 = lax.dot(xm, w1s[k], preferred_element_type=jnp.float32)
                hk = jnp.maximum(h1, 0.0).astype(jnp.bfloat16)
                acc = lax.dot(hk, w2s[k], preferred_element_type=jnp.float32)
                cur = pbuf[d, pl.ds(start, CAP), :].astype(jnp.float32)
                pbuf[d, pl.ds(start, CAP), :] = (cur + acc).astype(jnp.bfloat16)

        def rs_start(d):
            tz = lax.rem(mz + d, Z)
            r = pltpu.make_async_remote_copy(
                src_ref=pbuf.at[d],
                dst_ref=rbuf.at[d - 1],
                send_sem=rs_send.at[d - 1],
                recv_sem=rs_recv.at[d - 1],
                device_id=(mx, my, tz),
                device_id_type=pl.DeviceIdType.MESH,
            )
            r.start()
            return r

        ag_a.start()
        ag_b.start()
        expert_block(0)
        ag_a.wait_recv()
        ag_c.start()
        expert_block(Z - 1)
        rs = {Z - 1: rs_start(Z - 1)}
        ag_b.wait_recv()
        expert_block(1)
        rs[1] = rs_start(1)
        ag_c.wait_recv()
        expert_block(Z - 2)
        rs[Z - 2] = rs_start(Z - 2)

        for r in (ag_a, ag_b, ag_c):
            r.wait_send()
        for d in range(1, Z):
            rs[d].wait_recv()
        for d in range(1, Z):
            rs[d].wait_send()

        acc = pbuf[0].astype(jnp.float32)
        for j in range(Z - 1):
            acc = acc + rbuf[j].astype(jnp.float32)
        out_ref[...] = acc

    out_sorted = pl.pallas_call(
        body,
        out_shape=jax.ShapeDtypeStruct((T, D), jnp.float32),
        in_specs=[pl.BlockSpec(memory_space=pltpu.VMEM)] * 4,
        out_specs=pl.BlockSpec(memory_space=pltpu.VMEM),
        scratch_shapes=[
            pltpu.VMEM((Z, T, CW), jnp.bfloat16),
            pltpu.VMEM((Z, T, D), jnp.bfloat16),
            pltpu.VMEM((Z - 1, T, D), jnp.bfloat16),
            pltpu.SemaphoreType.DMA((Z - 1,)),
            pltpu.SemaphoreType.DMA((Z - 1,)),
            pltpu.SemaphoreType.DMA((Z - 1,)),
            pltpu.SemaphoreType.DMA((Z - 1,)),
        ],
        compiler_params=pltpu.CompilerParams(collective_id=0),
    )(xb, ab, w1b, w2b)
    return jnp.take(out_sorted, jnp.argsort(order), axis=0)


# device time: 35587 ns/iter; 1.9253x vs baseline; 1.9253x over previous
import jax
import jax.numpy as jnp
from jax import lax
from jax.experimental import pallas as pl
from jax.experimental.pallas import tpu as pltpu

Z = 4
ECAP = 96
PCAP = 2 * ECAP


def kernel(x, assign, W1, W2):
    T, D = x.shape
    E, _, F = W1.shape
    NE = Z * E

    eq = assign[:, None] == assign[None, :]
    before = jnp.arange(T)[:, None] > jnp.arange(T)[None, :]
    rank = jnp.sum(eq & before, axis=1)
    p8 = (
        (assign[None, None, :] == jnp.arange(NE)[:, None, None])
        & (rank[None, None, :] == jnp.arange(ECAP)[None, :, None])
    ).astype(jnp.bfloat16)
    ppair = p8.reshape(Z, PCAP, T)

    def body(x_ref, w1_ref, w2_ref, pp_ref, out_ref,
             sbuf, abuf, obuf, rbuf,
             fwd_send, fwd_recv, ret_send, ret_recv):
        mx = lax.axis_index("x")
        my = lax.axis_index("y")
        mz = lax.axis_index("z")

        barrier = pltpu.get_barrier_semaphore()
        for d in range(1, Z):
            peer = lax.rem(mz + d, Z)
            pl.semaphore_signal(
                barrier, inc=1,
                device_id=(mx, my, peer),
                device_id_type=pl.DeviceIdType.MESH,
            )
        pl.semaphore_wait(barrier, Z - 1)

        xl = x_ref[...].astype(jnp.bfloat16)

        fwd = {}
        for d in range(1, Z):
            tz = lax.rem(mz + d, Z)
            sbuf[d - 1] = lax.dot(
                pp_ref[tz], xl, preferred_element_type=jnp.float32
            ).astype(jnp.bfloat16)
            r = pltpu.make_async_remote_copy(
                src_ref=sbuf.at[d - 1],
                dst_ref=abuf.at[d - 1],
                send_sem=fwd_send.at[d - 1],
                recv_sem=fwd_recv.at[d - 1],
                device_id=(mx, my, tz),
                device_id_type=pl.DeviceIdType.MESH,
            )
            r.start()
            fwd[d] = r

        w1s = [w1_ref[k].astype(jnp.bfloat16) for k in range(E)]
        w2s = [w2_ref[k].astype(jnp.bfloat16) for k in range(E)]

        def ffn(win):
            outs = []
            for k in range(E):
                xk = win[k * ECAP:(k + 1) * ECAP, :]
                h1 = lax.dot(xk, w1s[k], preferred_element_type=jnp.float32)
                hk = jnp.maximum(h1, 0.0).astype(jnp.bfloat16)
                outs.append(
                    lax.dot(hk, w2s[k], preferred_element_type=jnp.float32)
                )
            return jnp.concatenate(outs, axis=0).astype(jnp.bfloat16)

        own = ffn(
            lax.dot(
                pp_ref[mz], xl, preferred_element_type=jnp.float32
            ).astype(jnp.bfloat16)
        )

        ret = {}
        for j in range(1, Z):
            wr = pltpu.make_async_remote_copy(
                src_ref=sbuf.at[0], dst_ref=abuf.at[j - 1],
                send_sem=fwd_send.at[0], recv_sem=fwd_recv.at[j - 1],
                device_id=(mx, my, mz),
                device_id_type=pl.DeviceIdType.MESH,
            )
            wr.wait_recv()
            obuf[j - 1] = ffn(abuf[j - 1])
            sz = lax.rem(mz + Z - j, Z)
            r = pltpu.make_async_remote_copy(
                src_ref=obuf.at[j - 1],
                dst_ref=rbuf.at[j - 1],
                send_sem=ret_send.at[j - 1],
                recv_sem=ret_recv.at[j - 1],
                device_id=(mx, my, sz),
                device_id_type=pl.DeviceIdType.MESH,
            )
            r.start()
            ret[j] = r

        acc = lax.dot_general(
            pp_ref[mz], own,
            dimension_numbers=(((0,), (0,)), ((), ())),
            preferred_element_type=jnp.float32,
        )
        for d in range(1, Z):
            wr = pltpu.make_async_remote_copy(
                src_ref=obuf.at[0], dst_ref=rbuf.at[d - 1],
                send_sem=ret_send.at[0], recv_sem=ret_recv.at[d - 1],
                device_id=(mx, my, mz),
                device_id_type=pl.DeviceIdType.MESH,
            )
            wr.wait_recv()
            tz = lax.rem(mz + d, Z)
            acc = acc + lax.dot_general(
                pp_ref[tz], rbuf[d - 1],
                dimension_numbers=(((0,), (0,)), ((), ())),
                preferred_element_type=jnp.float32,
            )
        out_ref[...] = acc

        for d in range(1, Z):
            fwd[d].wait_send()
            ret[d].wait_send()

    return pl.pallas_call(
        body,
        out_shape=jax.ShapeDtypeStruct((T, D), jnp.float32),
        in_specs=[pl.BlockSpec(memory_space=pltpu.VMEM)] * 4,
        out_specs=pl.BlockSpec(memory_space=pltpu.VMEM),
        scratch_shapes=[
            pltpu.VMEM((Z - 1, PCAP, D), jnp.bfloat16),
            pltpu.VMEM((Z - 1, PCAP, D), jnp.bfloat16),
            pltpu.VMEM((Z - 1, PCAP, D), jnp.bfloat16),
            pltpu.VMEM((Z - 1, PCAP, D), jnp.bfloat16),
            pltpu.SemaphoreType.DMA((Z - 1,)),
            pltpu.SemaphoreType.DMA((Z - 1,)),
            pltpu.SemaphoreType.DMA((Z - 1,)),
            pltpu.SemaphoreType.DMA((Z - 1,)),
        ],
        compiler_params=pltpu.CompilerParams(collective_id=0),
    )(x, W1, W2, ppair)


# device time: 32732 ns/iter; 2.0932x vs baseline; 1.0872x over previous
import jax
import jax.numpy as jnp
from jax import lax
from jax.experimental import pallas as pl
from jax.experimental.pallas import tpu as pltpu

Z = 4
ECAP = 80
PCAP = 2 * ECAP


def kernel(x, assign, W1, W2):
    T, D = x.shape
    E, _, F = W1.shape
    NE = Z * E

    eq = assign[:, None] == assign[None, :]
    before = jnp.arange(T)[:, None] > jnp.arange(T)[None, :]
    rank = jnp.sum(eq & before, axis=1)
    p8 = (
        (assign[None, None, :] == jnp.arange(NE)[:, None, None])
        & (rank[None, None, :] == jnp.arange(ECAP)[None, :, None])
    ).astype(jnp.bfloat16)
    ppair = p8.reshape(Z, PCAP, T)

    def body(x_ref, w1_ref, w2_ref, pp_ref, out_ref,
             sbuf, abuf, obuf, rbuf,
             fwd_send, fwd_recv, ret_send, ret_recv):
        mx = lax.axis_index("x")
        my = lax.axis_index("y")
        mz = lax.axis_index("z")

        barrier = pltpu.get_barrier_semaphore()
        for d in range(1, Z):
            peer = lax.rem(mz + d, Z)
            pl.semaphore_signal(
                barrier, inc=1,
                device_id=(mx, my, peer),
                device_id_type=pl.DeviceIdType.MESH,
            )
        pl.semaphore_wait(barrier, Z - 1)

        xl = x_ref[...].astype(jnp.bfloat16)

        fwd = {}
        for d in range(1, Z):
            tz = lax.rem(mz + d, Z)
            sbuf[d - 1] = lax.dot(
                pp_ref[tz], xl, preferred_element_type=jnp.float32
            ).astype(jnp.bfloat16)
            r = pltpu.make_async_remote_copy(
                src_ref=sbuf.at[d - 1],
                dst_ref=abuf.at[d - 1],
                send_sem=fwd_send.at[d - 1],
                recv_sem=fwd_recv.at[d - 1],
                device_id=(mx, my, tz),
                device_id_type=pl.DeviceIdType.MESH,
            )
            r.start()
            fwd[d] = r

        w1s = [w1_ref[k].astype(jnp.bfloat16) for k in range(E)]
        w2s = [w2_ref[k].astype(jnp.bfloat16) for k in range(E)]

        def ffn(win):
            outs = []
            for k in range(E):
                xk = win[k * ECAP:(k + 1) * ECAP, :]
                h1 = lax.dot(xk, w1s[k], preferred_element_type=jnp.float32)
                hk = jnp.maximum(h1, 0.0).astype(jnp.bfloat16)
                outs.append(
                    lax.dot(hk, w2s[k], preferred_element_type=jnp.float32)
                )
            return jnp.concatenate(outs, axis=0).astype(jnp.bfloat16)

        own = ffn(
            lax.dot(
                pp_ref[mz], xl, preferred_element_type=jnp.float32
            ).astype(jnp.bfloat16)
        )

        ret = {}
        for j in range(1, Z):
            wr = pltpu.make_async_remote_copy(
                src_ref=sbuf.at[0], dst_ref=abuf.at[j - 1],
                send_sem=fwd_send.at[0], recv_sem=fwd_recv.at[j - 1],
                device_id=(mx, my, mz),
                device_id_type=pl.DeviceIdType.MESH,
            )
            wr.wait_recv()
            obuf[j - 1] = ffn(abuf[j - 1])
            sz = lax.rem(mz + Z - j, Z)
            r = pltpu.make_async_remote_copy(
                src_ref=obuf.at[j - 1],
                dst_ref=rbuf.at[j - 1],
                send_sem=ret_send.at[j - 1],
                recv_sem=ret_recv.at[j - 1],
                device_id=(mx, my, sz),
                device_id_type=pl.DeviceIdType.MESH,
            )
            r.start()
            ret[j] = r

        acc = lax.dot_general(
            pp_ref[mz], own,
            dimension_numbers=(((0,), (0,)), ((), ())),
            preferred_element_type=jnp.float32,
        )
        for d in range(1, Z):
            wr = pltpu.make_async_remote_copy(
                src_ref=obuf.at[0], dst_ref=rbuf.at[d - 1],
                send_sem=ret_send.at[0], recv_sem=ret_recv.at[d - 1],
                device_id=(mx, my, mz),
                device_id_type=pl.DeviceIdType.MESH,
            )
            wr.wait_recv()
            tz = lax.rem(mz + d, Z)
            acc = acc + lax.dot_general(
                pp_ref[tz], rbuf[d - 1],
                dimension_numbers=(((0,), (0,)), ((), ())),
                preferred_element_type=jnp.float32,
            )
        out_ref[...] = acc

        for d in range(1, Z):
            fwd[d].wait_send()
            ret[d].wait_send()

    return pl.pallas_call(
        body,
        out_shape=jax.ShapeDtypeStruct((T, D), jnp.float32),
        in_specs=[pl.BlockSpec(memory_space=pltpu.VMEM)] * 4,
        out_specs=pl.BlockSpec(memory_space=pltpu.VMEM),
        scratch_shapes=[
            pltpu.VMEM((Z - 1, PCAP, D), jnp.bfloat16),
            pltpu.VMEM((Z - 1, PCAP, D), jnp.bfloat16),
            pltpu.VMEM((Z - 1, PCAP, D), jnp.bfloat16),
            pltpu.VMEM((Z - 1, PCAP, D), jnp.bfloat16),
            pltpu.SemaphoreType.DMA((Z - 1,)),
            pltpu.SemaphoreType.DMA((Z - 1,)),
            pltpu.SemaphoreType.DMA((Z - 1,)),
            pltpu.SemaphoreType.DMA((Z - 1,)),
        ],
        compiler_params=pltpu.CompilerParams(collective_id=0),
    )(x, W1, W2, ppair)
